# baseline (device time: 92086 ns/iter reference)
import jax
import jax.numpy as jnp
from jax import lax
from jax.experimental import pallas as pl
from jax.experimental.pallas import tpu as pltpu

N_DEV = 4
SUB = 4


def kernel(A, B):
    m, k = A.shape
    _, n = B.shape
    half = m // 2
    mc = half // N_DEV
    msc = mc // SUB

    out_dtype = jnp.bfloat16
    bf16 = jnp.bfloat16
    f32 = jnp.float32

    def body(a_ref, b_ref, out_ref, b_bf, acc_ref,
             send_r, send_l, recv_r, recv_l,
             r_send_sems, r_recv_sems, l_send_sems, l_recv_sems,
             agr_send_sems, agr_recv_sems, agl_send_sems, agl_recv_sems):
        my = lax.axis_index("i")
        left = lax.rem(my + N_DEV - 1, N_DEV)
        right = lax.rem(my + 1, N_DEV)

        barrier_sem = pltpu.get_barrier_semaphore()
        for nbr in [left, right]:
            pl.semaphore_signal(
                barrier_sem, inc=1,
                device_id=(nbr,), device_id_type=pl.DeviceIdType.MESH,
            )
        pl.semaphore_wait(barrier_sem, 2)

        b_bf[...] = b_ref[...].astype(bf16)

        def top_rows(c):
            return pl.ds(c * mc, mc)

        def bot_rows(c):
            return pl.ds(half + c * mc, mc)

        def dot_rows(rows):
            return jnp.dot(
                a_ref[rows, :].astype(bf16), b_bf[...],
                preferred_element_type=f32,
            )

        def mk_rs(s, j, sbuf, rbuf, ssem, rsem, dev):
            sub = pl.ds(j * msc, msc)
            return pltpu.make_async_remote_copy(
                src_ref=sbuf.at[s, sub, :], dst_ref=rbuf.at[s, sub, :],
                send_sem=ssem.at[s, j], recv_sem=rsem.at[s, j],
                device_id=(dev,), device_id_type=pl.DeviceIdType.MESH,
            )

        def mk_ag(t, j, chunk_rows_fn, chunk, ssem, rsem, dev):
            sub = pl.ds(chunk * mc + j * msc, msc)
            rows = (
                sub if chunk_rows_fn is top_rows
                else pl.ds(half + chunk * mc + j * msc, msc)
            )
            return pltpu.make_async_remote_copy(
                src_ref=out_ref.at[rows, :], dst_ref=out_ref.at[rows, :],
                send_sem=ssem.at[t, j], recv_sem=rsem.at[t, j],
                device_id=(dev,), device_id_type=pl.DeviceIdType.MESH,
            )


        send_r[0] = dot_rows(top_rows(lax.rem(my + N_DEV - 1, N_DEV))).astype(bf16)
        send_l[0] = dot_rows(bot_rows(lax.rem(my + 1, N_DEV))).astype(bf16)
        rs_r = {}
        rs_l = {}
        for j in range(SUB):
            rs_r[(0, j)] = mk_rs(0, j, send_r, recv_r, r_send_sems, r_recv_sems, right)
            rs_l[(0, j)] = mk_rs(0, j, send_l, recv_l, l_send_sems, l_recv_sems, left)
            rs_r[(0, j)].start()
            rs_l[(0, j)].start()

        ag_r = {}
        ag_l = {}
        for s in range(N_DEV - 1):
            rc_r = lax.rem(my + N_DEV - 2 - s, N_DEV)
            rc_l = lax.rem(my + 2 + s, N_DEV)
            acc_ref[top_rows(rc_r), :] = dot_rows(top_rows(rc_r))
            acc_ref[bot_rows(rc_l), :] = dot_rows(bot_rows(rc_l))
            for j in range(SUB):
                sub = pl.ds(j * msc, msc)
                sub_top = pl.ds(rc_r * mc + j * msc, msc)
                sub_bot = pl.ds(half + rc_l * mc + j * msc, msc)
                rs_r[(s, j)].wait()
                acc_rj = acc_ref[sub_top, :] + recv_r[s, sub, :].astype(f32)
                rs_l[(s, j)].wait()
                acc_lj = acc_ref[sub_bot, :] + recv_l[s, sub, :].astype(f32)
                if s < N_DEV - 2:
                    send_r[s + 1, sub, :] = acc_rj.astype(bf16)
                    send_l[s + 1, sub, :] = acc_lj.astype(bf16)
                    rs_r[(s + 1, j)] = mk_rs(
                        s + 1, j, send_r, recv_r, r_send_sems, r_recv_sems, right)
                    rs_l[(s + 1, j)] = mk_rs(
                        s + 1, j, send_l, recv_l, l_send_sems, l_recv_sems, left)
                    rs_r[(s + 1, j)].start()
                    rs_l[(s + 1, j)].start()
                else:
                    out_ref[sub_top, :] = jnp.maximum(acc_rj, 0.0).astype(out_dtype)
                    out_ref[sub_bot, :] = jnp.maximum(acc_lj, 0.0).astype(out_dtype)
                    ag_r[(0, j)] = mk_ag(
                        0, j, top_rows, my, agr_send_sems, agr_recv_sems, right)
                    ag_l[(0, j)] = mk_ag(
                        0, j, bot_rows, my, agl_send_sems, agl_recv_sems, left)
                    ag_r[(0, j)].start()
                    ag_l[(0, j)].start()

        for t in range(N_DEV - 1):
            fc_r = lax.rem(my + N_DEV - 1 - t, N_DEV)
            fc_l = lax.rem(my + 1 + t, N_DEV)
            for j in range(SUB):
                ag_r[(t, j)].wait()
                ag_l[(t, j)].wait()
                if t < N_DEV - 2:
                    ag_r[(t + 1, j)] = mk_ag(
                        t + 1, j, top_rows, fc_r, agr_send_sems, agr_recv_sems, right)
                    ag_l[(t + 1, j)] = mk_ag(
                        t + 1, j, bot_rows, fc_l, agl_send_sems, agl_recv_sems, left)
                    ag_r[(t + 1, j)].start()
                    ag_l[(t + 1, j)].start()

    return pl.pallas_call(
        body,
        out_shape=jax.ShapeDtypeStruct((m, n), out_dtype),
        in_specs=[
            pl.BlockSpec(memory_space=pltpu.VMEM),
            pl.BlockSpec(memory_space=pltpu.VMEM),
        ],
        out_specs=pl.BlockSpec(memory_space=pltpu.VMEM),
        scratch_shapes=[
            pltpu.VMEM((k, n), bf16),
            pltpu.VMEM((m, n), f32),
            pltpu.VMEM((N_DEV - 1, mc, n), bf16),
            pltpu.VMEM((N_DEV - 1, mc, n), bf16),
            pltpu.VMEM((N_DEV - 1, mc, n), bf16),
            pltpu.VMEM((N_DEV - 1, mc, n), bf16),
            pltpu.SemaphoreType.DMA((N_DEV - 1, SUB)),
            pltpu.SemaphoreType.DMA((N_DEV - 1, SUB)),
            pltpu.SemaphoreType.DMA((N_DEV - 1, SUB)),
            pltpu.SemaphoreType.DMA((N_DEV - 1, SUB)),
            pltpu.SemaphoreType.DMA((N_DEV - 1, SUB)),
            pltpu.SemaphoreType.DMA((N_DEV - 1, SUB)),
            pltpu.SemaphoreType.DMA((N_DEV - 1, SUB)),
            pltpu.SemaphoreType.DMA((N_DEV - 1, SUB)),
        ],
        compiler_params=pltpu.CompilerParams(
            collective_id=0,
            vmem_limit_bytes=100 * 1024 * 1024,
        ),
    )(A, B)


# device time: 90697 ns/iter; 1.0153x vs baseline; 1.0153x over previous
import jax
import jax.numpy as jnp
from jax import lax
from jax.experimental import pallas as pl
from jax.experimental.pallas import tpu as pltpu

N_DEV = 4
SUB = 2


def kernel(A, B):
    m, k = A.shape
    _, n = B.shape
    half = m // 2
    mc = half // N_DEV
    msc = mc // SUB

    out_dtype = jnp.bfloat16
    bf16 = jnp.bfloat16
    f32 = jnp.float32

    def body(a_ref, b_ref, out_ref, b_bf, acc_ref,
             send_r, send_l, recv_r, recv_l,
             r_send_sems, r_recv_sems, l_send_sems, l_recv_sems,
             agr_send_sems, agr_recv_sems, agl_send_sems, agl_recv_sems):
        my = lax.axis_index("i")
        left = lax.rem(my + N_DEV - 1, N_DEV)
        right = lax.rem(my + 1, N_DEV)

        barrier_sem = pltpu.get_barrier_semaphore()
        for nbr in [left, right]:
            pl.semaphore_signal(
                barrier_sem, inc=1,
                device_id=(nbr,), device_id_type=pl.DeviceIdType.MESH,
            )
        pl.semaphore_wait(barrier_sem, 2)

        b_bf[...] = b_ref[...].astype(bf16)

        def top_rows(c):
            return pl.ds(c * mc, mc)

        def bot_rows(c):
            return pl.ds(half + c * mc, mc)

        def dot_rows(rows):
            return jnp.dot(
                a_ref[rows, :].astype(bf16), b_bf[...],
                preferred_element_type=f32,
            )

        def mk_rs(s, j, sbuf, rbuf, ssem, rsem, dev):
            sub = pl.ds(j * msc, msc)
            return pltpu.make_async_remote_copy(
                src_ref=sbuf.at[s, sub, :], dst_ref=rbuf.at[s, sub, :],
                send_sem=ssem.at[s, j], recv_sem=rsem.at[s, j],
                device_id=(dev,), device_id_type=pl.DeviceIdType.MESH,
            )

        def mk_ag(t, j, chunk_rows_fn, chunk, ssem, rsem, dev):
            sub = pl.ds(chunk * mc + j * msc, msc)
            rows = (
                sub if chunk_rows_fn is top_rows
                else pl.ds(half + chunk * mc + j * msc, msc)
            )
            return pltpu.make_async_remote_copy(
                src_ref=out_ref.at[rows, :], dst_ref=out_ref.at[rows, :],
                send_sem=ssem.at[t, j], recv_sem=rsem.at[t, j],
                device_id=(dev,), device_id_type=pl.DeviceIdType.MESH,
            )


        sc_r0 = lax.rem(my + N_DEV - 1, N_DEV)
        sc_l0 = lax.rem(my + 1, N_DEV)
        rs_r = {}
        rs_l = {}
        for j in range(SUB):
            sub = pl.ds(j * msc, msc)
            send_r[0, sub, :] = dot_rows(
                pl.ds(sc_r0 * mc + j * msc, msc)).astype(bf16)
            rs_r[(0, j)] = mk_rs(0, j, send_r, recv_r, r_send_sems, r_recv_sems, right)
            rs_r[(0, j)].start()
            send_l[0, sub, :] = dot_rows(
                pl.ds(half + sc_l0 * mc + j * msc, msc)).astype(bf16)
            rs_l[(0, j)] = mk_rs(0, j, send_l, recv_l, l_send_sems, l_recv_sems, left)
            rs_l[(0, j)].start()

        ag_r = {}
        ag_l = {}
        for s in range(N_DEV - 1):
            rc_r = lax.rem(my + N_DEV - 2 - s, N_DEV)
            rc_l = lax.rem(my + 2 + s, N_DEV)
            acc_ref[top_rows(rc_r), :] = dot_rows(top_rows(rc_r))
            acc_ref[bot_rows(rc_l), :] = dot_rows(bot_rows(rc_l))
            for j in range(SUB):
                sub = pl.ds(j * msc, msc)
                sub_top = pl.ds(rc_r * mc + j * msc, msc)
                sub_bot = pl.ds(half + rc_l * mc + j * msc, msc)
                rs_r[(s, j)].wait()
                acc_rj = acc_ref[sub_top, :] + recv_r[s, sub, :].astype(f32)
                rs_l[(s, j)].wait()
                acc_lj = acc_ref[sub_bot, :] + recv_l[s, sub, :].astype(f32)
                if s < N_DEV - 2:
                    send_r[s + 1, sub, :] = acc_rj.astype(bf16)
                    send_l[s + 1, sub, :] = acc_lj.astype(bf16)
                    rs_r[(s + 1, j)] = mk_rs(
                        s + 1, j, send_r, recv_r, r_send_sems, r_recv_sems, right)
                    rs_l[(s + 1, j)] = mk_rs(
                        s + 1, j, send_l, recv_l, l_send_sems, l_recv_sems, left)
                    rs_r[(s + 1, j)].start()
                    rs_l[(s + 1, j)].start()
                else:
                    out_ref[sub_top, :] = jnp.maximum(acc_rj, 0.0).astype(out_dtype)
                    out_ref[sub_bot, :] = jnp.maximum(acc_lj, 0.0).astype(out_dtype)
                    ag_r[(0, j)] = mk_ag(
                        0, j, top_rows, my, agr_send_sems, agr_recv_sems, right)
                    ag_l[(0, j)] = mk_ag(
                        0, j, bot_rows, my, agl_send_sems, agl_recv_sems, left)
                    ag_r[(0, j)].start()
                    ag_l[(0, j)].start()

        for t in range(N_DEV - 1):
            fc_r = lax.rem(my + N_DEV - 1 - t, N_DEV)
            fc_l = lax.rem(my + 1 + t, N_DEV)
            for j in range(SUB):
                ag_r[(t, j)].wait()
                ag_l[(t, j)].wait()
                if t < N_DEV - 2:
                    ag_r[(t + 1, j)] = mk_ag(
                        t + 1, j, top_rows, fc_r, agr_send_sems, agr_recv_sems, right)
                    ag_l[(t + 1, j)] = mk_ag(
                        t + 1, j, bot_rows, fc_l, agl_send_sems, agl_recv_sems, left)
                    ag_r[(t + 1, j)].start()
                    ag_l[(t + 1, j)].start()

    return pl.pallas_call(
        body,
        out_shape=jax.ShapeDtypeStruct((m, n), out_dtype),
        in_specs=[
            pl.BlockSpec(memory_space=pltpu.VMEM),
            pl.BlockSpec(memory_space=pltpu.VMEM),
        ],
        out_specs=pl.BlockSpec(memory_space=pltpu.VMEM),
        scratch_shapes=[
            pltpu.VMEM((k, n), bf16),
            pltpu.VMEM((m, n), f32),
            pltpu.VMEM((N_DEV - 1, mc, n), bf16),
            pltpu.VMEM((N_DEV - 1, mc, n), bf16),
            pltpu.VMEM((N_DEV - 1, mc, n), bf16),
            pltpu.VMEM((N_DEV - 1, mc, n), bf16),
            pltpu.SemaphoreType.DMA((N_DEV - 1, SUB)),
            pltpu.SemaphoreType.DMA((N_DEV - 1, SUB)),
            pltpu.SemaphoreType.DMA((N_DEV - 1, SUB)),
            pltpu.SemaphoreType.DMA((N_DEV - 1, SUB)),
            pltpu.SemaphoreType.DMA((N_DEV - 1, SUB)),
            pltpu.SemaphoreType.DMA((N_DEV - 1, SUB)),
            pltpu.SemaphoreType.DMA((N_DEV - 1, SUB)),
            pltpu.SemaphoreType.DMA((N_DEV - 1, SUB)),
        ],
        compiler_params=pltpu.CompilerParams(
            collective_id=0,
            vmem_limit_bytes=100 * 1024 * 1024,
        ),
    )(A, B)


# device time: 90650 ns/iter; 1.0158x vs baseline; 1.0005x over previous
import jax
import jax.numpy as jnp
from jax import lax
from jax.experimental import pallas as pl
from jax.experimental.pallas import tpu as pltpu

N_DEV = 4
SUB = 2


def kernel(A, B):
    m, k = A.shape
    _, n = B.shape
    half = m // 2
    mc = half // N_DEV
    msc = mc // SUB

    out_dtype = jnp.bfloat16
    bf16 = jnp.bfloat16
    f32 = jnp.float32

    def body(a_ref, b_ref, out_ref, b_bf, part_r, part_l,
             send_r, send_l, recv_r, recv_l,
             r_send_sems, r_recv_sems, l_send_sems, l_recv_sems,
             agr_send_sems, agr_recv_sems, agl_send_sems, agl_recv_sems):
        my = lax.axis_index("i")
        left = lax.rem(my + N_DEV - 1, N_DEV)
        right = lax.rem(my + 1, N_DEV)

        barrier_sem = pltpu.get_barrier_semaphore()
        for nbr in [left, right]:
            pl.semaphore_signal(
                barrier_sem, inc=1,
                device_id=(nbr,), device_id_type=pl.DeviceIdType.MESH,
            )
        pl.semaphore_wait(barrier_sem, 2)

        b_bf[...] = b_ref[...].astype(bf16)

        def top_rows(c):
            return pl.ds(c * mc, mc)

        def bot_rows(c):
            return pl.ds(half + c * mc, mc)

        def dot_rows(rows):
            return jnp.dot(
                a_ref[rows, :].astype(bf16), b_bf[...],
                preferred_element_type=f32,
            ).astype(bf16)

        def mk_rs(s, j, sbuf, rbuf, ssem, rsem, dev):
            sub = pl.ds(j * msc, msc)
            return pltpu.make_async_remote_copy(
                src_ref=sbuf.at[s, sub, :], dst_ref=rbuf.at[s, sub, :],
                send_sem=ssem.at[s, j], recv_sem=rsem.at[s, j],
                device_id=(dev,), device_id_type=pl.DeviceIdType.MESH,
            )

        def mk_ag(t, j, chunk_rows_fn, chunk, ssem, rsem, dev):
            sub = pl.ds(chunk * mc + j * msc, msc)
            rows = (
                sub if chunk_rows_fn is top_rows
                else pl.ds(half + chunk * mc + j * msc, msc)
            )
            return pltpu.make_async_remote_copy(
                src_ref=out_ref.at[rows, :], dst_ref=out_ref.at[rows, :],
                send_sem=ssem.at[t, j], recv_sem=rsem.at[t, j],
                device_id=(dev,), device_id_type=pl.DeviceIdType.MESH,
            )


        sc_r0 = lax.rem(my + N_DEV - 1, N_DEV)
        sc_l0 = lax.rem(my + 1, N_DEV)
        rs_r = {}
        rs_l = {}
        for j in range(SUB):
            sub = pl.ds(j * msc, msc)
            send_r[0, sub, :] = dot_rows(pl.ds(sc_r0 * mc + j * msc, msc))
            rs_r[(0, j)] = mk_rs(0, j, send_r, recv_r, r_send_sems, r_recv_sems, right)
            rs_r[(0, j)].start()
            send_l[0, sub, :] = dot_rows(pl.ds(half + sc_l0 * mc + j * msc, msc))
            rs_l[(0, j)] = mk_rs(0, j, send_l, recv_l, l_send_sems, l_recv_sems, left)
            rs_l[(0, j)].start()

        ag_r = {}
        ag_l = {}
        for s in range(N_DEV - 1):
            rc_r = lax.rem(my + N_DEV - 2 - s, N_DEV)
            rc_l = lax.rem(my + 2 + s, N_DEV)
            part_r[s] = dot_rows(top_rows(rc_r))
            part_l[s] = dot_rows(bot_rows(rc_l))
            for j in range(SUB):
                sub = pl.ds(j * msc, msc)
                sub_top = pl.ds(rc_r * mc + j * msc, msc)
                sub_bot = pl.ds(half + rc_l * mc + j * msc, msc)
                rs_r[(s, j)].wait()
                acc_rj = part_r[s, sub, :] + recv_r[s, sub, :]
                rs_l[(s, j)].wait()
                acc_lj = part_l[s, sub, :] + recv_l[s, sub, :]
                if s < N_DEV - 2:
                    send_r[s + 1, sub, :] = acc_rj
                    send_l[s + 1, sub, :] = acc_lj
                    rs_r[(s + 1, j)] = mk_rs(
                        s + 1, j, send_r, recv_r, r_send_sems, r_recv_sems, right)
                    rs_l[(s + 1, j)] = mk_rs(
                        s + 1, j, send_l, recv_l, l_send_sems, l_recv_sems, left)
                    rs_r[(s + 1, j)].start()
                    rs_l[(s + 1, j)].start()
                else:
                    zero = jnp.zeros((), bf16)
                    out_ref[sub_top, :] = jnp.maximum(acc_rj, zero)
                    out_ref[sub_bot, :] = jnp.maximum(acc_lj, zero)
                    ag_r[(0, j)] = mk_ag(
                        0, j, top_rows, my, agr_send_sems, agr_recv_sems, right)
                    ag_l[(0, j)] = mk_ag(
                        0, j, bot_rows, my, agl_send_sems, agl_recv_sems, left)
                    ag_r[(0, j)].start()
                    ag_l[(0, j)].start()

        for t in range(N_DEV - 1):
            fc_r = lax.rem(my + N_DEV - 1 - t, N_DEV)
            fc_l = lax.rem(my + 1 + t, N_DEV)
            for j in range(SUB):
                ag_r[(t, j)].wait()
                ag_l[(t, j)].wait()
                if t < N_DEV - 2:
                    ag_r[(t + 1, j)] = mk_ag(
                        t + 1, j, top_rows, fc_r, agr_send_sems, agr_recv_sems, right)
                    ag_l[(t + 1, j)] = mk_ag(
                        t + 1, j, bot_rows, fc_l, agl_send_sems, agl_recv_sems, left)
                    ag_r[(t + 1, j)].start()
                    ag_l[(t + 1, j)].start()

    return pl.pallas_call(
        body,
        out_shape=jax.ShapeDtypeStruct((m, n), out_dtype),
        in_specs=[
            pl.BlockSpec(memory_space=pltpu.VMEM),
            pl.BlockSpec(memory_space=pltpu.VMEM),
        ],
        out_specs=pl.BlockSpec(memory_space=pltpu.VMEM),
        scratch_shapes=[
            pltpu.VMEM((k, n), bf16),
            pltpu.VMEM((N_DEV - 1, mc, n), bf16),
            pltpu.VMEM((N_DEV - 1, mc, n), bf16),
            pltpu.VMEM((N_DEV - 1, mc, n), bf16),
            pltpu.VMEM((N_DEV - 1, mc, n), bf16),
            pltpu.VMEM((N_DEV - 1, mc, n), bf16),
            pltpu.VMEM((N_DEV - 1, mc, n), bf16),
            pltpu.SemaphoreType.DMA((N_DEV - 1, SUB)),
            pltpu.SemaphoreType.DMA((N_DEV - 1, SUB)),
            pltpu.SemaphoreType.DMA((N_DEV - 1, SUB)),
            pltpu.SemaphoreType.DMA((N_DEV - 1, SUB)),
            pltpu.SemaphoreType.DMA((N_DEV - 1, SUB)),
            pltpu.SemaphoreType.DMA((N_DEV - 1, SUB)),
            pltpu.SemaphoreType.DMA((N_DEV - 1, SUB)),
            pltpu.SemaphoreType.DMA((N_DEV - 1, SUB)),
        ],
        compiler_params=pltpu.CompilerParams(
            collective_id=0,
            vmem_limit_bytes=100 * 1024 * 1024,
        ),
    )(A, B)


# device time: 87774 ns/iter; 1.0491x vs baseline; 1.0328x over previous
import jax
import jax.numpy as jnp
from jax import lax
from jax.experimental import pallas as pl
from jax.experimental.pallas import tpu as pltpu

N_DEV = 4
SUB = 2


def kernel(A, B):
    m, k = A.shape
    _, n = B.shape
    half = m // 2
    mc = half // N_DEV
    msc = mc // SUB

    out_dtype = jnp.bfloat16
    bf16 = jnp.bfloat16
    f32 = jnp.float32

    def body(a_ref, b_ref, out_ref, b_bf, part_r, part_l, g_ref,
             send_r, send_l, recv_r, recv_l,
             r_send_sems, r_recv_sems, l_send_sems, l_recv_sems,
             agr_send_sems, agr_recv_sems, agl_send_sems, agl_recv_sems,
             hbm_top_sems, hbm_bot_sems):
        my = lax.axis_index("i")
        left = lax.rem(my + N_DEV - 1, N_DEV)
        right = lax.rem(my + 1, N_DEV)

        barrier_sem = pltpu.get_barrier_semaphore()
        for nbr in [left, right]:
            pl.semaphore_signal(
                barrier_sem, inc=1,
                device_id=(nbr,), device_id_type=pl.DeviceIdType.MESH,
            )
        pl.semaphore_wait(barrier_sem, 2)

        b_bf[...] = b_ref[...].astype(bf16)

        def top_rows(c):
            return pl.ds(c * mc, mc)

        def bot_rows(c):
            return pl.ds(half + c * mc, mc)

        def dot_rows(rows):
            return jnp.dot(
                a_ref[rows, :].astype(bf16), b_bf[...],
                preferred_element_type=f32,
            ).astype(bf16)

        def mk_rs(s, j, sbuf, rbuf, ssem, rsem, dev):
            sub = pl.ds(j * msc, msc)
            return pltpu.make_async_remote_copy(
                src_ref=sbuf.at[s, sub, :], dst_ref=rbuf.at[s, sub, :],
                send_sem=ssem.at[s, j], recv_sem=rsem.at[s, j],
                device_id=(dev,), device_id_type=pl.DeviceIdType.MESH,
            )

        def mk_ag(t, j, chunk_rows_fn, chunk, ssem, rsem, dev):
            sub = pl.ds(chunk * mc + j * msc, msc)
            rows = (
                sub if chunk_rows_fn is top_rows
                else pl.ds(half + chunk * mc + j * msc, msc)
            )
            return pltpu.make_async_remote_copy(
                src_ref=g_ref.at[rows, :], dst_ref=g_ref.at[rows, :],
                send_sem=ssem.at[t, j], recv_sem=rsem.at[t, j],
                device_id=(dev,), device_id_type=pl.DeviceIdType.MESH,
            )

        hbm_copies = []

        def start_hbm_copy(rows_slice, slot, sems):
            cp = pltpu.make_async_copy(
                g_ref.at[rows_slice, :], out_ref.at[rows_slice, :],
                sems.at[slot],
            )
            cp.start()
            hbm_copies.append(cp)


        sc_r0 = lax.rem(my + N_DEV - 1, N_DEV)
        sc_l0 = lax.rem(my + 1, N_DEV)
        rs_r = {}
        rs_l = {}
        for j in range(SUB):
            sub = pl.ds(j * msc, msc)
            send_r[0, sub, :] = dot_rows(pl.ds(sc_r0 * mc + j * msc, msc))
            rs_r[(0, j)] = mk_rs(0, j, send_r, recv_r, r_send_sems, r_recv_sems, right)
            rs_r[(0, j)].start()
            send_l[0, sub, :] = dot_rows(pl.ds(half + sc_l0 * mc + j * msc, msc))
            rs_l[(0, j)] = mk_rs(0, j, send_l, recv_l, l_send_sems, l_recv_sems, left)
            rs_l[(0, j)].start()

        ag_r = {}
        ag_l = {}
        for s in range(N_DEV - 1):
            rc_r = lax.rem(my + N_DEV - 2 - s, N_DEV)
            rc_l = lax.rem(my + 2 + s, N_DEV)
            part_r[s] = dot_rows(top_rows(rc_r))
            part_l[s] = dot_rows(bot_rows(rc_l))
            for j in range(SUB):
                sub = pl.ds(j * msc, msc)
                sub_top = pl.ds(rc_r * mc + j * msc, msc)
                sub_bot = pl.ds(half + rc_l * mc + j * msc, msc)
                rs_r[(s, j)].wait()
                acc_rj = part_r[s, sub, :] + recv_r[s, sub, :]
                rs_l[(s, j)].wait()
                acc_lj = part_l[s, sub, :] + recv_l[s, sub, :]
                if s < N_DEV - 2:
                    send_r[s + 1, sub, :] = acc_rj
                    send_l[s + 1, sub, :] = acc_lj
                    rs_r[(s + 1, j)] = mk_rs(
                        s + 1, j, send_r, recv_r, r_send_sems, r_recv_sems, right)
                    rs_l[(s + 1, j)] = mk_rs(
                        s + 1, j, send_l, recv_l, l_send_sems, l_recv_sems, left)
                    rs_r[(s + 1, j)].start()
                    rs_l[(s + 1, j)].start()
                else:
                    zero = jnp.zeros((), bf16)
                    g_ref[sub_top, :] = jnp.maximum(acc_rj, zero)
                    g_ref[sub_bot, :] = jnp.maximum(acc_lj, zero)
                    ag_r[(0, j)] = mk_ag(
                        0, j, top_rows, my, agr_send_sems, agr_recv_sems, right)
                    ag_l[(0, j)] = mk_ag(
                        0, j, bot_rows, my, agl_send_sems, agl_recv_sems, left)
                    ag_r[(0, j)].start()
                    ag_l[(0, j)].start()
                    if j == SUB - 1:
                        start_hbm_copy(top_rows(my), N_DEV - 1, hbm_top_sems)
                        start_hbm_copy(bot_rows(my), N_DEV - 1, hbm_bot_sems)

        for t in range(N_DEV - 1):
            fc_r = lax.rem(my + N_DEV - 1 - t, N_DEV)
            fc_l = lax.rem(my + 1 + t, N_DEV)
            for j in range(SUB):
                ag_r[(t, j)].wait()
                ag_l[(t, j)].wait()
                if t < N_DEV - 2:
                    ag_r[(t + 1, j)] = mk_ag(
                        t + 1, j, top_rows, fc_r, agr_send_sems, agr_recv_sems, right)
                    ag_l[(t + 1, j)] = mk_ag(
                        t + 1, j, bot_rows, fc_l, agl_send_sems, agl_recv_sems, left)
                    ag_r[(t + 1, j)].start()
                    ag_l[(t + 1, j)].start()
                if j == SUB - 1:
                    start_hbm_copy(top_rows(fc_r), t, hbm_top_sems)
                    start_hbm_copy(bot_rows(fc_l), t, hbm_bot_sems)

        for cp in hbm_copies:
            cp.wait()

    return pl.pallas_call(
        body,
        out_shape=jax.ShapeDtypeStruct((m, n), out_dtype),
        in_specs=[
            pl.BlockSpec(memory_space=pltpu.VMEM),
            pl.BlockSpec(memory_space=pltpu.VMEM),
        ],
        out_specs=pl.BlockSpec(memory_space=pltpu.MemorySpace.HBM),
        scratch_shapes=[
            pltpu.VMEM((k, n), bf16),
            pltpu.VMEM((N_DEV - 1, mc, n), bf16),
            pltpu.VMEM((N_DEV - 1, mc, n), bf16),
            pltpu.VMEM((m, n), bf16),
            pltpu.VMEM((N_DEV - 1, mc, n), bf16),
            pltpu.VMEM((N_DEV - 1, mc, n), bf16),
            pltpu.VMEM((N_DEV - 1, mc, n), bf16),
            pltpu.VMEM((N_DEV - 1, mc, n), bf16),
            pltpu.SemaphoreType.DMA((N_DEV - 1, SUB)),
            pltpu.SemaphoreType.DMA((N_DEV - 1, SUB)),
            pltpu.SemaphoreType.DMA((N_DEV - 1, SUB)),
            pltpu.SemaphoreType.DMA((N_DEV - 1, SUB)),
            pltpu.SemaphoreType.DMA((N_DEV - 1, SUB)),
            pltpu.SemaphoreType.DMA((N_DEV - 1, SUB)),
            pltpu.SemaphoreType.DMA((N_DEV - 1, SUB)),
            pltpu.SemaphoreType.DMA((N_DEV - 1, SUB)),
            pltpu.SemaphoreType.DMA((N_DEV,)),
            pltpu.SemaphoreType.DMA((N_DEV,)),
        ],
        compiler_params=pltpu.CompilerParams(
            collective_id=0,
            vmem_limit_bytes=100 * 1024 * 1024,
        ),
    )(A, B)
